# baseline (speedup 1.0000x reference)
import functools
import jax
import jax.numpy as jnp
from jax import lax
from jax.experimental import pallas as pl
from jax.experimental.pallas import tpu as pltpu
from jax.experimental.pallas import tpu_sc as plsc

NN = 10000
EE = 320000
DD = 128
NC = 2
NS = 16
NW = NC * NS
NPAD = 10240
TPW = NPAD // NS
RPT = NN // NS

CH = 80
NCHROWS = EE // CH
CPW = NCHROWS // NW
CH1 = 125
CPT1 = EE // CH1 // NS


def _mesh():
    return plsc.VectorSubcoreMesh(core_axis_name="c", subcore_axis_name="s")


def _k1_body(ei_hbm, ones_hbm, z640_hbm, degs_hbm,
             out_sp, in_sp, idx3, ones_v, s0, s1, s2, s3):
    c = lax.axis_index("c")
    s = lax.axis_index("s")
    wk = c * NS + s
    sems = [s0, s1, s2, s3]
    pltpu.sync_copy(z640_hbm, out_sp.at[pl.ds(s * TPW, TPW)])
    pltpu.sync_copy(z640_hbm, in_sp.at[pl.ds(s * TPW, TPW)])
    pltpu.sync_copy(ones_hbm, ones_v)
    pltpu.sync_copy(ei_hbm.at[pl.ds(wk * CPW, CPW)], idx3)
    plsc.subcore_barrier()

    def issue(j, b):
        pltpu.async_copy(ones_v, out_sp.at[idx3.at[j, 0]], sems[b], add=True)
        pltpu.async_copy(ones_v, in_sp.at[idx3.at[j, 1]], sems[b], add=True)

    def drain(j, b):
        pltpu.make_async_copy(ones_v, out_sp.at[idx3.at[j, 0]], sems[b]).wait()
        pltpu.make_async_copy(ones_v, in_sp.at[idx3.at[j, 1]], sems[b]).wait()

    def grp(g, carry):
        for b in range(4):
            j = g * 4 + b

            @pl.when(g >= 1)
            def _():
                drain(j - 4, b)

            issue(j, b)
        return carry

    lax.fori_loop(0, CPW // 4, grp, 0)
    drain(CPW - 5, 0)
    issue(CPW - 1, 0)
    for b in range(4):
        j = CPW - 4 + b
        drain(j, j % 4)

    plsc.subcore_barrier()
    pltpu.sync_copy(out_sp.at[pl.ds(s * TPW, TPW)],
                    degs_hbm.at[c, 0, pl.ds(s * TPW, TPW)])
    pltpu.sync_copy(in_sp.at[pl.ds(s * TPW, TPW)],
                    degs_hbm.at[c, 1, pl.ds(s * TPW, TPW)])


_k1 = pl.kernel(
    _k1_body,
    out_type=jax.ShapeDtypeStruct((NC, 2, NPAD), jnp.float32),
    mesh=_mesh(),
    scratch_types=[
        pltpu.VMEM_SHARED((NPAD,), jnp.float32),
        pltpu.VMEM_SHARED((NPAD,), jnp.float32),
        pltpu.VMEM((CPW, 2, CH), jnp.int32),
        pltpu.VMEM((CH,), jnp.float32),
        pltpu.SemaphoreType.DMA,
        pltpu.SemaphoreType.DMA,
        pltpu.SemaphoreType.DMA,
        pltpu.SemaphoreType.DMA,
    ],
)


def _k2_body(x_ref, win_ref, bin_ref, degs_ref, g1_ref, norms_ref):
    h = jnp.tanh(jnp.dot(x_ref[...], win_ref[...],
                         preferred_element_type=jnp.float32) + bin_ref[...])
    degs = degs_ref[...]
    nout = lax.rsqrt(jnp.maximum(degs[:, 0:1], 1.0))
    nin = lax.rsqrt(jnp.maximum(degs[:, 1:2], 1.0))
    g1_ref[...] = h * nout
    norms_ref[...] = jnp.concatenate([nin, nout], axis=1)


def _k2(x, w_in, b_in, degs):
    blk = 1000
    grid = NN // blk
    return pl.pallas_call(
        _k2_body,
        grid=(grid,),
        in_specs=[
            pl.BlockSpec((blk, DD), lambda i: (i, 0)),
            pl.BlockSpec((DD, DD), lambda i: (0, 0)),
            pl.BlockSpec((1, DD), lambda i: (0, 0)),
            pl.BlockSpec((blk, 2), lambda i: (i, 0)),
        ],
        out_specs=[
            pl.BlockSpec((blk, DD), lambda i: (i, 0)),
            pl.BlockSpec((blk, 2), lambda i: (i, 0)),
        ],
        out_shape=[
            jax.ShapeDtypeStruct((NN, DD), jnp.float32),
            jax.ShapeDtypeStruct((NN, 2), jnp.float32),
        ],
    )(x, w_in, b_in, degs)


def _k3_body(ei_hbm, g1_hbm, nin_hbm, zrows_hbm, z640_hbm,
             aggp_hbm, cp_hbm,
             agg_sp, c_sp,
             e0, e1, e2, e3,
             r0, r1, r2, r3, n0, n1, n2, n3,
             is0, is1, is2, is3, gs0, gs1, gs2, gs3,
             ss0, ss1, ss2, ss3):
    c = lax.axis_index("c")
    s = lax.axis_index("s")
    base = (c * NS + s) * CPW
    eb = [e0, e1, e2, e3]
    rows = [r0, r1, r2, r3]
    nrm = [n0, n1, n2, n3]
    isem = [is0, is1, is2, is3]
    gsem = [gs0, gs1, gs2, gs3]
    ssem = [ss0, ss1, ss2, ss3]

    pltpu.sync_copy(zrows_hbm, agg_sp.at[pl.ds(s * TPW, TPW)])
    pltpu.sync_copy(z640_hbm, c_sp.at[pl.ds(s * TPW, TPW)])
    plsc.subcore_barrier()

    def load_idx(j, t):
        pltpu.async_copy(ei_hbm.at[base + j], eb[t], isem[t])

    def wait_idx(j, t):
        pltpu.make_async_copy(ei_hbm.at[base + j], eb[t], isem[t]).wait()

    def issue_gather(t):
        pltpu.async_copy(g1_hbm.at[eb[t].at[0]], rows[t], gsem[t])
        pltpu.async_copy(nin_hbm.at[eb[t].at[1]], nrm[t], gsem[t])

    def wait_gather(t):
        pltpu.make_async_copy(g1_hbm.at[eb[t].at[0]], rows[t], gsem[t]).wait()
        pltpu.make_async_copy(nin_hbm.at[eb[t].at[1]], nrm[t], gsem[t]).wait()

    def issue_scatter(t):
        pltpu.async_copy(rows[t], agg_sp.at[eb[t].at[1]], ssem[t], add=True)
        pltpu.async_copy(nrm[t], c_sp.at[eb[t].at[0]], ssem[t], add=True)

    def wait_scatter(t):
        pltpu.make_async_copy(rows[t], agg_sp.at[eb[t].at[1]], ssem[t]).wait()
        pltpu.make_async_copy(nrm[t], c_sp.at[eb[t].at[0]], ssem[t]).wait()

    for t in range(3):
        load_idx(t, t)
    wait_idx(0, 0)
    issue_gather(0)
    wait_idx(1, 1)
    issue_gather(1)

    LAST = CPW - 1

    def grp(g, carry):
        for u in range(4):
            j = 4 * g + u
            wait_gather(u)
            issue_scatter(u)

            um = (u - 1) % 4

            @pl.when(j >= 1)
            def _():
                wait_scatter(um)

            un = (u + 2) % 4

            @pl.when(j + 2 <= LAST)
            def _():
                wait_idx(j + 2, un)
                issue_gather(un)

            u3 = (u + 3) % 4

            @pl.when(j + 3 <= LAST)
            def _():
                load_idx(j + 3, u3)
        return carry

    lax.fori_loop(0, CPW // 4, grp, 0)
    wait_gather(0)
    issue_scatter(0)
    wait_scatter(3)
    wait_scatter(0)

    plsc.subcore_barrier()
    pltpu.sync_copy(agg_sp.at[pl.ds(s * TPW, TPW)],
                    aggp_hbm.at[c, pl.ds(s * TPW, TPW)])
    pltpu.sync_copy(c_sp.at[pl.ds(s * TPW, TPW)],
                    cp_hbm.at[c, pl.ds(s * TPW, TPW)])


_k3 = pl.kernel(
    _k3_body,
    out_type=[
        jax.ShapeDtypeStruct((NC, NPAD, DD), jnp.float32),
        jax.ShapeDtypeStruct((NC, NPAD), jnp.float32),
    ],
    mesh=_mesh(),
    scratch_types=[
        pltpu.VMEM_SHARED((NPAD, DD), jnp.float32),
        pltpu.VMEM_SHARED((NPAD,), jnp.float32),
        pltpu.VMEM((2, CH), jnp.int32),
        pltpu.VMEM((2, CH), jnp.int32),
        pltpu.VMEM((2, CH), jnp.int32),
        pltpu.VMEM((2, CH), jnp.int32),
        pltpu.VMEM((CH, DD), jnp.float32),
        pltpu.VMEM((CH, DD), jnp.float32),
        pltpu.VMEM((CH, DD), jnp.float32),
        pltpu.VMEM((CH, DD), jnp.float32),
        pltpu.VMEM((CH,), jnp.float32),
        pltpu.VMEM((CH,), jnp.float32),
        pltpu.VMEM((CH,), jnp.float32),
        pltpu.VMEM((CH,), jnp.float32),
        pltpu.SemaphoreType.DMA,
        pltpu.SemaphoreType.DMA,
        pltpu.SemaphoreType.DMA,
        pltpu.SemaphoreType.DMA,
        pltpu.SemaphoreType.DMA,
        pltpu.SemaphoreType.DMA,
        pltpu.SemaphoreType.DMA,
        pltpu.SemaphoreType.DMA,
        pltpu.SemaphoreType.DMA,
        pltpu.SemaphoreType.DMA,
        pltpu.SemaphoreType.DMA,
        pltpu.SemaphoreType.DMA,
    ],
)


def _k4_body(aggp_ref, cw_ref, normsp_ref, w0_ref, b0_ref, w1_ref, b1_ref,
             out_ref, acc_ref):
    i = pl.program_id(0)

    @pl.when(i == 0)
    def _():
        acc_ref[...] = jnp.zeros_like(acc_ref)

    agg = aggp_ref[0] + aggp_ref[1]
    nin = normsp_ref[:, 0:1]
    nout = normsp_ref[:, 1:2]
    u = agg * nin
    z = jnp.maximum(jnp.dot(u, w0_ref[...],
                            preferred_element_type=jnp.float32) + b0_ref[...],
                    0.0)
    w = (cw_ref[:, 0:1] + cw_ref[:, 1:2]) * nout
    acc_ref[...] += jnp.sum(w * z, axis=0, keepdims=True)

    @pl.when(i == pl.num_programs(0) - 1)
    def _():
        out_ref[...] = (jnp.dot(acc_ref[...], w1_ref[...],
                                preferred_element_type=jnp.float32)
                        + float(NN) * b1_ref[...])


def _k4(aggp, cw, normsp, w0, b0, w1, b1):
    blk = 1024
    grid = NPAD // blk
    return pl.pallas_call(
        _k4_body,
        grid=(grid,),
        in_specs=[
            pl.BlockSpec((NC, blk, DD), lambda i: (0, i, 0)),
            pl.BlockSpec((blk, NC), lambda i: (i, 0)),
            pl.BlockSpec((blk, 2), lambda i: (i, 0)),
            pl.BlockSpec((DD, DD), lambda i: (0, 0)),
            pl.BlockSpec((1, DD), lambda i: (0, 0)),
            pl.BlockSpec((DD, DD), lambda i: (0, 0)),
            pl.BlockSpec((1, DD), lambda i: (0, 0)),
        ],
        out_specs=pl.BlockSpec((1, DD), lambda i: (0, 0)),
        out_shape=jax.ShapeDtypeStruct((1, DD), jnp.float32),
        scratch_shapes=[pltpu.VMEM((1, DD), jnp.float32)],
    )(aggp, cw, normsp, w0, b0, w1, b1)


@jax.jit
def kernel(x, edge_index, W_in, b_in, W0, b0, W1, b1):
    e2d = jnp.stack([edge_index[0].reshape(NCHROWS, CH),
                     edge_index[1].reshape(NCHROWS, CH)],
                    axis=1)
    ones80 = jnp.ones((CH,), jnp.float32)
    z640 = jnp.zeros((TPW,), jnp.float32)
    zrows = jnp.zeros((TPW, DD), jnp.float32)

    degs2 = _k1(e2d, ones80, z640)
    dsum = degs2[0] + degs2[1]
    degs = jnp.transpose(dsum[:, :NN])
    g1, norms = _k2(x, W_in, b_in.reshape(1, DD), degs)
    nin1d = norms[:, 0]
    aggp, cp = _k3(e2d, g1, nin1d, zrows, z640)
    cw = jnp.transpose(cp)
    normsp = jnp.concatenate(
        [norms, jnp.zeros((NPAD - NN, 2), jnp.float32)], axis=0)
    return _k4(aggp, cw, normsp, W0, b0.reshape(1, DD),
               W1, b1.reshape(1, DD))


# speedup vs baseline: 1.0190x; 1.0190x over previous
import functools
import jax
import jax.numpy as jnp
from jax import lax
from jax.experimental import pallas as pl
from jax.experimental.pallas import tpu as pltpu
from jax.experimental.pallas import tpu_sc as plsc

NN = 10000
EE = 320000
DD = 128
NC = 2
NS = 16
NW = NC * NS
NPAD = 10240
TPW = NPAD // NS
RPT = NN // NS

CH = 80
NCHROWS = EE // CH
CPW = NCHROWS // NW
CH1 = 125
CPT1 = EE // CH1 // NS


def _mesh():
    return plsc.VectorSubcoreMesh(core_axis_name="c", subcore_axis_name="s")


def _k1_body(src2d, dst2d, ones_hbm, z640_hbm, degs_hbm,
             deg_sp, idx2, ones_v, s0, s1, s2, s3):
    c = lax.axis_index("c")
    s = lax.axis_index("s")
    sems = [s0, s1, s2, s3]
    pltpu.sync_copy(z640_hbm, deg_sp.at[pl.ds(s * TPW, TPW)])
    pltpu.sync_copy(ones_hbm, ones_v)

    @pl.when(c == 0)
    def _():
        pltpu.sync_copy(src2d.at[pl.ds(s * CPT1, CPT1)], idx2)

    @pl.when(c == 1)
    def _():
        pltpu.sync_copy(dst2d.at[pl.ds(s * CPT1, CPT1)], idx2)

    plsc.subcore_barrier()

    def grp(g, carry):
        for b in range(4):
            j = g * 4 + b

            @pl.when(g >= 1)
            def _():
                pltpu.make_async_copy(
                    ones_v, deg_sp.at[idx2.at[j - 4]], sems[b]).wait()

            pltpu.async_copy(ones_v, deg_sp.at[idx2.at[j]], sems[b], add=True)
        return carry

    lax.fori_loop(0, CPT1 // 4, grp, 0)
    for b in range(4):
        j = CPT1 - 4 + b
        pltpu.make_async_copy(ones_v, deg_sp.at[idx2.at[j]], sems[b]).wait()

    plsc.subcore_barrier()
    pltpu.sync_copy(deg_sp.at[pl.ds(s * TPW, TPW)],
                    degs_hbm.at[c, pl.ds(s * TPW, TPW)])


_k1 = pl.kernel(
    _k1_body,
    out_type=jax.ShapeDtypeStruct((NC, NPAD), jnp.float32),
    mesh=_mesh(),
    scratch_types=[
        pltpu.VMEM_SHARED((NPAD,), jnp.float32),
        pltpu.VMEM((CPT1, CH1), jnp.int32),
        pltpu.VMEM((CH1,), jnp.float32),
        pltpu.SemaphoreType.DMA,
        pltpu.SemaphoreType.DMA,
        pltpu.SemaphoreType.DMA,
        pltpu.SemaphoreType.DMA,
    ],
)


def _k2_body(x_ref, win_ref, bin_ref, degs_ref, g1_ref, norms_ref):
    h = jnp.tanh(jnp.dot(x_ref[...], win_ref[...],
                         preferred_element_type=jnp.float32) + bin_ref[...])
    degs = degs_ref[...]
    nout = lax.rsqrt(jnp.maximum(degs[:, 0:1], 1.0))
    nin = lax.rsqrt(jnp.maximum(degs[:, 1:2], 1.0))
    g1_ref[...] = h * nout
    norms_ref[...] = jnp.concatenate([nin, nout], axis=1)


def _k2(x, w_in, b_in, degs):
    blk = 1000
    grid = NN // blk
    return pl.pallas_call(
        _k2_body,
        grid=(grid,),
        in_specs=[
            pl.BlockSpec((blk, DD), lambda i: (i, 0)),
            pl.BlockSpec((DD, DD), lambda i: (0, 0)),
            pl.BlockSpec((1, DD), lambda i: (0, 0)),
            pl.BlockSpec((blk, 2), lambda i: (i, 0)),
        ],
        out_specs=[
            pl.BlockSpec((blk, DD), lambda i: (i, 0)),
            pl.BlockSpec((blk, 2), lambda i: (i, 0)),
        ],
        out_shape=[
            jax.ShapeDtypeStruct((NN, DD), jnp.float32),
            jax.ShapeDtypeStruct((NN, 2), jnp.float32),
        ],
    )(x, w_in, b_in, degs)


def _k3_body(src_hbm, dst_hbm, g1_hbm, nin_hbm, zrows_hbm, z640_hbm,
             aggp_hbm, cp_hbm,
             agg_sp, c_sp,
             sb0, sb1, sb2, sb3, db0, db1, db2, db3,
             r0, r1, r2, r3, n0, n1, n2, n3,
             is0, is1, is2, is3, gs0, gs1, gs2, gs3,
             ss0, ss1, ss2, ss3):
    c = lax.axis_index("c")
    s = lax.axis_index("s")
    base = (c * NS + s) * CPW
    srcb = [sb0, sb1, sb2, sb3]
    dstb = [db0, db1, db2, db3]
    rows = [r0, r1, r2, r3]
    nrm = [n0, n1, n2, n3]
    isem = [is0, is1, is2, is3]
    gsem = [gs0, gs1, gs2, gs3]
    ssem = [ss0, ss1, ss2, ss3]

    pltpu.sync_copy(zrows_hbm, agg_sp.at[pl.ds(s * TPW, TPW)])
    pltpu.sync_copy(z640_hbm, c_sp.at[pl.ds(s * TPW, TPW)])
    plsc.subcore_barrier()

    def load_idx(j, t):
        off = (base + j) * CH
        pltpu.async_copy(src_hbm.at[pl.ds(off, CH)], srcb[t], isem[t])
        pltpu.async_copy(dst_hbm.at[pl.ds(off, CH)], dstb[t], isem[t])

    def wait_idx(j, t):
        off = (base + j) * CH
        pltpu.make_async_copy(
            src_hbm.at[pl.ds(off, CH)], srcb[t], isem[t]).wait()
        pltpu.make_async_copy(
            dst_hbm.at[pl.ds(off, CH)], dstb[t], isem[t]).wait()

    def issue_gather(t):
        pltpu.async_copy(g1_hbm.at[srcb[t]], rows[t], gsem[t])
        pltpu.async_copy(nin_hbm.at[dstb[t]], nrm[t], gsem[t])

    def wait_gather(t):
        pltpu.make_async_copy(g1_hbm.at[srcb[t]], rows[t], gsem[t]).wait()
        pltpu.make_async_copy(nin_hbm.at[dstb[t]], nrm[t], gsem[t]).wait()

    def issue_scatter(t):
        pltpu.async_copy(rows[t], agg_sp.at[dstb[t]], ssem[t], add=True)
        pltpu.async_copy(nrm[t], c_sp.at[srcb[t]], ssem[t], add=True)

    def wait_scatter(t):
        pltpu.make_async_copy(rows[t], agg_sp.at[dstb[t]], ssem[t]).wait()
        pltpu.make_async_copy(nrm[t], c_sp.at[srcb[t]], ssem[t]).wait()

    for t in range(3):
        load_idx(t, t)
    wait_idx(0, 0)
    issue_gather(0)
    wait_idx(1, 1)
    issue_gather(1)

    LAST = CPW - 1

    def grp(g, carry):
        for u in range(4):
            j = 4 * g + u
            wait_gather(u)
            issue_scatter(u)

            um = (u - 1) % 4

            @pl.when(j >= 1)
            def _():
                wait_scatter(um)

            un = (u + 2) % 4

            @pl.when(j + 2 <= LAST)
            def _():
                wait_idx(j + 2, un)
                issue_gather(un)

            u3 = (u + 3) % 4

            @pl.when(j + 3 <= LAST)
            def _():
                load_idx(j + 3, u3)
        return carry

    lax.fori_loop(0, CPW // 4, grp, 0)
    wait_gather(0)
    issue_scatter(0)
    wait_scatter(3)
    wait_scatter(0)

    plsc.subcore_barrier()
    pltpu.sync_copy(agg_sp.at[pl.ds(s * TPW, TPW)],
                    aggp_hbm.at[c, pl.ds(s * TPW, TPW)])
    pltpu.sync_copy(c_sp.at[pl.ds(s * TPW, TPW)],
                    cp_hbm.at[c, pl.ds(s * TPW, TPW)])


_k3 = pl.kernel(
    _k3_body,
    out_type=[
        jax.ShapeDtypeStruct((NC, NPAD, DD), jnp.float32),
        jax.ShapeDtypeStruct((NC, NPAD), jnp.float32),
    ],
    mesh=_mesh(),
    scratch_types=[
        pltpu.VMEM_SHARED((NPAD, DD), jnp.float32),
        pltpu.VMEM_SHARED((NPAD,), jnp.float32),
        pltpu.VMEM((CH,), jnp.int32),
        pltpu.VMEM((CH,), jnp.int32),
        pltpu.VMEM((CH,), jnp.int32),
        pltpu.VMEM((CH,), jnp.int32),
        pltpu.VMEM((CH,), jnp.int32),
        pltpu.VMEM((CH,), jnp.int32),
        pltpu.VMEM((CH,), jnp.int32),
        pltpu.VMEM((CH,), jnp.int32),
        pltpu.VMEM((CH, DD), jnp.float32),
        pltpu.VMEM((CH, DD), jnp.float32),
        pltpu.VMEM((CH, DD), jnp.float32),
        pltpu.VMEM((CH, DD), jnp.float32),
        pltpu.VMEM((CH,), jnp.float32),
        pltpu.VMEM((CH,), jnp.float32),
        pltpu.VMEM((CH,), jnp.float32),
        pltpu.VMEM((CH,), jnp.float32),
        pltpu.SemaphoreType.DMA,
        pltpu.SemaphoreType.DMA,
        pltpu.SemaphoreType.DMA,
        pltpu.SemaphoreType.DMA,
        pltpu.SemaphoreType.DMA,
        pltpu.SemaphoreType.DMA,
        pltpu.SemaphoreType.DMA,
        pltpu.SemaphoreType.DMA,
        pltpu.SemaphoreType.DMA,
        pltpu.SemaphoreType.DMA,
        pltpu.SemaphoreType.DMA,
        pltpu.SemaphoreType.DMA,
    ],
)


def _k4_body(aggp_ref, cw_ref, normsp_ref, w0_ref, b0_ref, w1_ref, b1_ref,
             out_ref, acc_ref):
    i = pl.program_id(0)

    @pl.when(i == 0)
    def _():
        acc_ref[...] = jnp.zeros_like(acc_ref)

    agg = aggp_ref[0] + aggp_ref[1]
    nin = normsp_ref[:, 0:1]
    nout = normsp_ref[:, 1:2]
    u = agg * nin
    z = jnp.maximum(jnp.dot(u, w0_ref[...],
                            preferred_element_type=jnp.float32) + b0_ref[...],
                    0.0)
    w = (cw_ref[:, 0:1] + cw_ref[:, 1:2]) * nout
    acc_ref[...] += jnp.sum(w * z, axis=0, keepdims=True)

    @pl.when(i == pl.num_programs(0) - 1)
    def _():
        out_ref[...] = (jnp.dot(acc_ref[...], w1_ref[...],
                                preferred_element_type=jnp.float32)
                        + float(NN) * b1_ref[...])


def _k4(aggp, cw, normsp, w0, b0, w1, b1):
    blk = 1024
    grid = NPAD // blk
    return pl.pallas_call(
        _k4_body,
        grid=(grid,),
        in_specs=[
            pl.BlockSpec((NC, blk, DD), lambda i: (0, i, 0)),
            pl.BlockSpec((blk, NC), lambda i: (i, 0)),
            pl.BlockSpec((blk, 2), lambda i: (i, 0)),
            pl.BlockSpec((DD, DD), lambda i: (0, 0)),
            pl.BlockSpec((1, DD), lambda i: (0, 0)),
            pl.BlockSpec((DD, DD), lambda i: (0, 0)),
            pl.BlockSpec((1, DD), lambda i: (0, 0)),
        ],
        out_specs=pl.BlockSpec((1, DD), lambda i: (0, 0)),
        out_shape=jax.ShapeDtypeStruct((1, DD), jnp.float32),
        scratch_shapes=[pltpu.VMEM((1, DD), jnp.float32)],
    )(aggp, cw, normsp, w0, b0, w1, b1)


@jax.jit
def kernel(x, edge_index, W_in, b_in, W0, b0, W1, b1):
    src = edge_index[0]
    dst = edge_index[1]
    src2d = src.reshape(EE // CH1, CH1)
    dst2d = dst.reshape(EE // CH1, CH1)
    ones125 = jnp.ones((CH1,), jnp.float32)
    z640 = jnp.zeros((TPW,), jnp.float32)
    zrows = jnp.zeros((TPW, DD), jnp.float32)

    degs2 = _k1(src2d, dst2d, ones125, z640)
    degs = jnp.transpose(degs2[:, :NN])
    g1, norms = _k2(x, W_in, b_in.reshape(1, DD), degs)
    nin1d = norms[:, 0]
    aggp, cp = _k3(src, dst, g1, nin1d, zrows, z640)
    cw = jnp.transpose(cp)
    normsp = jnp.concatenate(
        [norms, jnp.zeros((NPAD - NN, 2), jnp.float32)], axis=0)
    return _k4(aggp, cw, normsp, W0, b0.reshape(1, DD),
               W1, b1.reshape(1, DD))
